# Initial kernel scaffold; baseline (speedup 1.0000x reference)
import jax
import jax.numpy as jnp
from jax.experimental import pallas as pl


def kernel(x, w):
    raise NotImplementedError("write your pallas kernel here")


# speedup vs baseline: 40.5561x; 40.5561x over previous
import jax
import jax.numpy as jnp
from jax import lax
from jax.experimental import pallas as pl
from jax.experimental.pallas import tpu as pltpu
from jax.experimental.pallas import tpu_sc as plsc

_N = 4
_SEG = 64
_IN = 128
_OUT = 64
_B = 1024
_ROW = _N * _OUT
_NC = 2
_NS = 16
_NW = _NC * _NS
_BPW = _B // _NW
_CH = _IN // 16


def _sc_body(x_hbm, t_hbm, out_hbm, x_v, idx_v, cw_v, g0, g1, out_v, sem0, sem1):
    wid = lax.axis_index("s") * _NC + lax.axis_index("c")
    base = wid * _BPW
    pltpu.sync_copy(x_hbm.at[pl.ds(base, _BPW)], x_v)

    ivec64 = lax.iota(jnp.int32, 16) * _SEG

    @pl.loop(0, _BPW)
    def _prep(b):
        for c in range(_CH):
            sl = pl.ds(c * 16, 16)
            xv = x_v[b, sl]
            u = 32.0 * xv + 32.0
            seg = jnp.minimum(jnp.maximum(u.astype(jnp.int32), 0), _SEG - 1)
            idx_v[b, sl] = ivec64 + (c * 16 * _SEG) + seg
            t = 2.0 * (u - seg.astype(jnp.float32)) - 1.0
            ta = t * t - (1.0 / 9.0)
            tb = t * t - 1.0
            cw_v[b, 0, sl] = (-9.0 / 16.0) * ta * (t - 1.0)
            cw_v[b, 1, sl] = (27.0 / 16.0) * tb * (t - 1.0 / 3.0)
            cw_v[b, 2, sl] = (-27.0 / 16.0) * tb * (t + 1.0 / 3.0)
            cw_v[b, 3, sl] = (9.0 / 16.0) * ta * (t + 1.0)

    bufs = (g0, g1)
    sems = (sem0, sem1)

    def issue(b, which):
        pltpu.async_copy(t_hbm.at[idx_v.at[b]], bufs[which], sems[which])

    def wait(b, which):
        pltpu.make_async_copy(t_hbm.at[idx_v.at[b]], bufs[which], sems[which]).wait()

    def compute(b, which):
        g = bufs[which]

        def body(c, accs):
            a0, a1, a2, a3 = accs
            cwv = [cw_v[b, k, pl.ds(c * 16, 16)] for k in range(_N)]
            for j in range(16):
                i = c * 16 + j
                for k in range(_N):
                    ck = cwv[k][j]
                    a0 = a0 + ck * g[i, pl.ds(k * _OUT + 0, 16)]
                    a1 = a1 + ck * g[i, pl.ds(k * _OUT + 16, 16)]
                    a2 = a2 + ck * g[i, pl.ds(k * _OUT + 32, 16)]
                    a3 = a3 + ck * g[i, pl.ds(k * _OUT + 48, 16)]
            return a0, a1, a2, a3

        zero = jnp.zeros((16,), jnp.float32)
        acc = lax.fori_loop(0, _CH, body, (zero, zero, zero, zero))
        for o4 in range(4):
            out_v[b, pl.ds(o4 * 16, 16)] = acc[o4]

    issue(0, 0)

    @pl.loop(0, _BPW // 2)
    def _main(p):
        b0 = 2 * p
        issue(b0 + 1, 1)
        wait(b0, 0)
        compute(b0, 0)

        @pl.when(p < _BPW // 2 - 1)
        def _():
            issue(b0 + 2, 0)

        wait(b0 + 1, 1)
        compute(b0 + 1, 1)

    pltpu.sync_copy(out_v, out_hbm.at[pl.ds(base, _BPW)])


_sc_call = pl.kernel(
    _sc_body,
    out_type=jax.ShapeDtypeStruct((_B, _OUT), jnp.float32),
    mesh=plsc.VectorSubcoreMesh(core_axis_name="c", subcore_axis_name="s"),
    scratch_types=[
        pltpu.VMEM((_BPW, _IN), jnp.float32),
        pltpu.VMEM((_BPW, _IN), jnp.int32),
        pltpu.VMEM((_BPW, _N, _IN), jnp.float32),
        pltpu.VMEM((_IN, _ROW), jnp.float32),
        pltpu.VMEM((_IN, _ROW), jnp.float32),
        pltpu.VMEM((_BPW, _OUT), jnp.float32),
        pltpu.SemaphoreType.DMA,
        pltpu.SemaphoreType.DMA,
    ],
)


def kernel(x, w):
    w2 = jnp.transpose(w, (1, 2, 0))
    u = w2[:, 0 : 3 * _SEG, :].reshape(_IN, _SEG, 3 * _OUT)
    v = w2[:, 1 : 3 * _SEG + 1, :].reshape(_IN, _SEG, 3, _OUT)[:, :, 2, :]
    table = jnp.concatenate([u, v], axis=2).reshape(_IN * _SEG, _ROW)
    return _sc_call(x, table)


# speedup vs baseline: 42.1842x; 1.0401x over previous
import jax
import jax.numpy as jnp
from jax import lax
from jax.experimental import pallas as pl
from jax.experimental.pallas import tpu as pltpu
from jax.experimental.pallas import tpu_sc as plsc

_N = 4
_SEG = 64
_IN = 128
_OUT = 64
_B = 1024
_ROW = _N * _OUT
_NC = 2
_NS = 16
_NW = _NC * _NS
_BPW = _B // _NW
_CH = _IN // 16


def _sc_body(x_hbm, t_hbm, out_hbm, x_v, idx_v, cw_v, g0, g1, out_v, sem0, sem1):
    wid = lax.axis_index("s") * _NC + lax.axis_index("c")
    base = wid * _BPW
    pltpu.sync_copy(x_hbm.at[pl.ds(base, _BPW)], x_v)

    ivec64 = lax.iota(jnp.int32, 16) * _SEG

    @pl.loop(0, _BPW)
    def _prep(b):
        for c in range(_CH):
            sl = pl.ds(c * 16, 16)
            xv = x_v[b, sl]
            u = 32.0 * xv + 32.0
            seg = jnp.minimum(jnp.maximum(u.astype(jnp.int32), 0), _SEG - 1)
            idx_v[b, sl] = ivec64 + (c * 16 * _SEG) + seg
            t = 2.0 * (u - seg.astype(jnp.float32)) - 1.0
            ta = t * t - (1.0 / 9.0)
            tb = t * t - 1.0
            cw_v[b, 0, sl] = (-9.0 / 16.0) * ta * (t - 1.0)
            cw_v[b, 1, sl] = (27.0 / 16.0) * tb * (t - 1.0 / 3.0)
            cw_v[b, 2, sl] = (-27.0 / 16.0) * tb * (t + 1.0 / 3.0)
            cw_v[b, 3, sl] = (9.0 / 16.0) * ta * (t + 1.0)

    bufs = (g0, g1)
    sems = (sem0, sem1)

    def issue(b, which):
        pltpu.async_copy(t_hbm.at[idx_v.at[b]], bufs[which], sems[which])

    def wait(b, which):
        pltpu.make_async_copy(t_hbm.at[idx_v.at[b]], bufs[which], sems[which]).wait()

    def compute(b, which):
        g = bufs[which]
        zero = jnp.zeros((16,), jnp.float32)
        for o4 in range(4):
            out_v[b, pl.ds(o4 * 16, 16)] = zero

        def body(c, carry):
            cwv = [cw_v[b, k, pl.ds(c * 16, 16)] for k in range(_N)]
            for j in range(16):
                i = c * 16 + j
                ck = [cwv[k][j] for k in range(_N)]
                for o4 in range(4):
                    t0 = ck[0] * g[i, pl.ds(0 * _OUT + o4 * 16, 16)]
                    t1 = ck[1] * g[i, pl.ds(1 * _OUT + o4 * 16, 16)]
                    t2 = ck[2] * g[i, pl.ds(2 * _OUT + o4 * 16, 16)]
                    t3 = ck[3] * g[i, pl.ds(3 * _OUT + o4 * 16, 16)]
                    plsc.addupdate(out_v.at[b, pl.ds(o4 * 16, 16)],
                                   (t0 + t1) + (t2 + t3))
            return carry

        lax.fori_loop(0, _CH, body, 0)

    issue(0, 0)

    @pl.loop(0, _BPW // 2)
    def _main(p):
        b0 = 2 * p
        issue(b0 + 1, 1)
        wait(b0, 0)
        compute(b0, 0)

        @pl.when(p < _BPW // 2 - 1)
        def _():
            issue(b0 + 2, 0)

        wait(b0 + 1, 1)
        compute(b0 + 1, 1)

    pltpu.sync_copy(out_v, out_hbm.at[pl.ds(base, _BPW)])


_sc_call = pl.kernel(
    _sc_body,
    out_type=jax.ShapeDtypeStruct((_B, _OUT), jnp.float32),
    mesh=plsc.VectorSubcoreMesh(core_axis_name="c", subcore_axis_name="s"),
    scratch_types=[
        pltpu.VMEM((_BPW, _IN), jnp.float32),
        pltpu.VMEM((_BPW, _IN), jnp.int32),
        pltpu.VMEM((_BPW, _N, _IN), jnp.float32),
        pltpu.VMEM((_IN, _ROW), jnp.float32),
        pltpu.VMEM((_IN, _ROW), jnp.float32),
        pltpu.VMEM((_BPW, _OUT), jnp.float32),
        pltpu.SemaphoreType.DMA,
        pltpu.SemaphoreType.DMA,
    ],
)


def kernel(x, w):
    w2 = jnp.transpose(w, (1, 2, 0))
    u = w2[:, 0 : 3 * _SEG, :].reshape(_IN, _SEG, 3 * _OUT)
    v = w2[:, 1 : 3 * _SEG + 1, :].reshape(_IN, _SEG, 3, _OUT)[:, :, 2, :]
    table = jnp.concatenate([u, v], axis=2).reshape(_IN * _SEG, _ROW)
    return _sc_call(x, table)


# speedup vs baseline: 53.3921x; 1.2657x over previous
import jax
import jax.numpy as jnp
import numpy as np
from jax import lax
from jax.experimental import pallas as pl
from jax.experimental.pallas import tpu as pltpu
from jax.experimental.pallas import tpu_sc as plsc

_N = 4
_SEG = 64
_IN = 128
_OUT = 64
_B = 1024
_ROW = _N * _OUT
_NC = 2
_NS = 16
_NW = _NC * _NS
_BPW = _B // _NW
_CH = _IN // 16
_NBUF = 4


def _col_perm() -> np.ndarray:
    p = np.empty((_ROW,), np.int32)
    for m in range(8):
        k, m2 = divmod(m, 2)
        for t in range(16):
            p[32 * m + 2 * t] = 64 * k + 32 * m2 + t
            p[32 * m + 2 * t + 1] = 64 * k + 32 * m2 + 16 + t
    return p


def _sc_body(x_hbm, t_hbm, out_hbm, x_v, idx_v, cw_v, gb, out_v, sems):
    wid = lax.axis_index("s") * _NC + lax.axis_index("c")
    base = wid * _BPW
    pltpu.sync_copy(x_hbm.at[pl.ds(base, _BPW)], x_v)

    ivec64 = lax.iota(jnp.int32, 16) * _SEG

    @pl.loop(0, _BPW)
    def _prep(b):
        for c in range(_CH):
            sl = pl.ds(c * 16, 16)
            xv = x_v[b, sl]
            u = 32.0 * xv + 32.0
            seg = jnp.minimum(jnp.maximum(u.astype(jnp.int32), 0), _SEG - 1)
            idx_v[b, sl] = ivec64 + (c * 16 * _SEG) + seg
            t = 2.0 * (u - seg.astype(jnp.float32)) - 1.0
            ta = t * t - (1.0 / 9.0)
            tb = t * t - 1.0
            cw_v[b, 0, sl] = (-9.0 / 16.0) * ta * (t - 1.0)
            cw_v[b, 1, sl] = (27.0 / 16.0) * tb * (t - 1.0 / 3.0)
            cw_v[b, 2, sl] = (-27.0 / 16.0) * tb * (t + 1.0 / 3.0)
            cw_v[b, 3, sl] = (9.0 / 16.0) * ta * (t + 1.0)

    def issue(b, which):
        pltpu.async_copy(t_hbm.at[idx_v.at[b]], gb[which], sems[which])

    def wait(b, which):
        pltpu.make_async_copy(t_hbm.at[idx_v.at[b]], gb[which], sems[which]).wait()

    def compute(b, which):
        g = gb[which]
        zero = jnp.zeros((16,), jnp.float32)
        for o4 in range(4):
            out_v[b, pl.ds(o4 * 16, 16)] = zero

        def body(c, carry):
            cwv = [cw_v[b, k, pl.ds(c * 16, 16)] for k in range(_N)]
            for j in range(16):
                i = c * 16 + j
                ck = [cwv[k][j] for k in range(_N)]
                acc = [None] * 4
                for k in range(_N):
                    for m2 in range(2):
                        vi = g[i, pl.ds(16 * (2 * k + m2), 16)]
                        ve = lax.bitcast_convert_type(vi << 16, jnp.float32)
                        vo = lax.bitcast_convert_type(
                            vi & jnp.int32(-65536), jnp.float32)
                        pe = ck[k] * ve
                        po = ck[k] * vo
                        ge, go = 2 * m2, 2 * m2 + 1
                        acc[ge] = pe if acc[ge] is None else acc[ge] + pe
                        acc[go] = po if acc[go] is None else acc[go] + po
                for o4 in range(4):
                    plsc.addupdate(out_v.at[b, pl.ds(o4 * 16, 16)], acc[o4])
            return carry

        lax.fori_loop(0, _CH, body, 0)

    for b0 in range(_NBUF - 1):
        issue(b0, b0)

    @pl.loop(0, _BPW // _NBUF)
    def _main(q):
        b0 = q * _NBUF
        for r in range(_NBUF):
            b = b0 + r
            wait(b, r)
            nxt = b + (_NBUF - 1)

            @pl.when(nxt < _BPW)
            def _():
                issue(nxt, (r + _NBUF - 1) % _NBUF)

            compute(b, r)

    pltpu.sync_copy(out_v, out_hbm.at[pl.ds(base, _BPW)])


def _make_call():
    def body(x_hbm, t_hbm, out_hbm, x_v, idx_v, cw_v,
             g0, g1, g2, g3, out_v, s0, s1, s2, s3):
        _sc_body(x_hbm, t_hbm, out_hbm, x_v, idx_v, cw_v,
                 (g0, g1, g2, g3), out_v, (s0, s1, s2, s3))

    return pl.kernel(
        body,
        out_type=jax.ShapeDtypeStruct((_B, _OUT), jnp.float32),
        mesh=plsc.VectorSubcoreMesh(core_axis_name="c", subcore_axis_name="s"),
        scratch_types=[
            pltpu.VMEM((_BPW, _IN), jnp.float32),
            pltpu.VMEM((_BPW, _IN), jnp.int32),
            pltpu.VMEM((_BPW, _N, _IN), jnp.float32),
            pltpu.VMEM((_IN, _ROW // 2), jnp.int32),
            pltpu.VMEM((_IN, _ROW // 2), jnp.int32),
            pltpu.VMEM((_IN, _ROW // 2), jnp.int32),
            pltpu.VMEM((_IN, _ROW // 2), jnp.int32),
            pltpu.VMEM((_BPW, _OUT), jnp.float32),
            pltpu.SemaphoreType.DMA,
            pltpu.SemaphoreType.DMA,
            pltpu.SemaphoreType.DMA,
            pltpu.SemaphoreType.DMA,
        ],
    )


_sc_call = _make_call()
_PERM = _col_perm()


def kernel(x, w):
    w2 = jnp.transpose(w, (1, 2, 0))
    u = w2[:, 0 : 3 * _SEG, :].reshape(_IN, _SEG, 3 * _OUT)
    v = w2[:, 1 : 3 * _SEG + 1, :].reshape(_IN, _SEG, 3, _OUT)[:, :, 2, :]
    table = jnp.concatenate([u, v], axis=2).reshape(_IN * _SEG, _ROW)
    table = table[:, _PERM].astype(jnp.bfloat16)
    table = lax.bitcast_convert_type(
        table.reshape(_IN * _SEG, _ROW // 2, 2), jnp.int32)
    return _sc_call(x, table)


# speedup vs baseline: 70.7073x; 1.3243x over previous
import jax
import jax.numpy as jnp
from jax import lax
from jax.experimental import pallas as pl
from jax.experimental.pallas import tpu as pltpu
from jax.experimental.pallas import tpu_sc as plsc

_N = 4
_SEG = 64
_IN = 128
_OUT = 64
_B = 1024
_ROW = _N * _OUT
_NC = 2
_NS = 16
_NW = _NC * _NS
_BPW = _B // _NW
_CH = _IN // 16
_NBUF = 4


_IPB = 8


def _tc_build(w_ref, sel_ref, out_ref):
    for di in range(_IPB):
        w_i = w_ref[:, di, :]
        a = lax.dot_general(sel_ref[...], w_i,
                            (((1,), (1,)), ((), ())),
                            preferred_element_type=jnp.float32)
        b32 = lax.bitcast_convert_type(a, jnp.int32)
        rb = b32 + jnp.int32(0x7FFF) + ((b32 >> 16) & 1)
        bfb = lax.shift_right_logical(rb, 16)
        for m in range(8):
            k, m2 = divmod(m, 2)
            ak = bfb[64 * k : 64 * (k + 1), :]
            lo = ak[:, 32 * m2 : 32 * m2 + 16]
            hi = ak[:, 32 * m2 + 16 : 32 * m2 + 32]
            out_ref[pl.ds(di * _SEG, _SEG), 16 * m : 16 * (m + 1)] = (
                hi << 16) | lo


_tc_table = pl.pallas_call(
    _tc_build,
    grid=(_IN // _IPB,),
    in_specs=[
        pl.BlockSpec((_OUT, _IPB, 193), lambda i: (0, i, 0)),
        pl.BlockSpec((_N * _SEG, 193), lambda i: (0, 0)),
    ],
    out_specs=pl.BlockSpec((_IPB * _SEG, _ROW // 2), lambda i: (i, 0)),
    out_shape=jax.ShapeDtypeStruct((_IN * _SEG, _ROW // 2), jnp.int32),
)


def _sel_matrix():
    l = jnp.arange(193, dtype=jnp.int32)[None, :]
    ks = jnp.arange(_N * _SEG, dtype=jnp.int32)
    tgt = (ks % _SEG) * 3 + ks // _SEG
    return (l == tgt[:, None]).astype(jnp.float32)


def _sc_body(x_hbm, t_hbm, out_hbm, x_v, idx_v, cw_v, gb, out_v, sems):
    wid = lax.axis_index("s") * _NC + lax.axis_index("c")
    base = wid * _BPW
    pltpu.sync_copy(x_hbm.at[pl.ds(base, _BPW)], x_v)

    ivec64 = lax.iota(jnp.int32, 16) * _SEG

    @pl.loop(0, _BPW)
    def _prep(b):
        for c in range(_CH):
            sl = pl.ds(c * 16, 16)
            xv = x_v[b, sl]
            u = 32.0 * xv + 32.0
            seg = jnp.minimum(jnp.maximum(u.astype(jnp.int32), 0), _SEG - 1)
            idx_v[b, sl] = ivec64 + (c * 16 * _SEG) + seg
            t = 2.0 * (u - seg.astype(jnp.float32)) - 1.0
            ta = t * t - (1.0 / 9.0)
            tb = t * t - 1.0
            cw_v[b, 0, sl] = (-9.0 / 16.0) * ta * (t - 1.0)
            cw_v[b, 1, sl] = (27.0 / 16.0) * tb * (t - 1.0 / 3.0)
            cw_v[b, 2, sl] = (-27.0 / 16.0) * tb * (t + 1.0 / 3.0)
            cw_v[b, 3, sl] = (9.0 / 16.0) * ta * (t + 1.0)

    def issue(b, which):
        pltpu.async_copy(t_hbm.at[idx_v.at[b]], gb[which], sems[which])

    def wait(b, which):
        pltpu.make_async_copy(t_hbm.at[idx_v.at[b]], gb[which], sems[which]).wait()

    def compute(b, which):
        g = gb[which]
        zero = jnp.zeros((16,), jnp.float32)
        for o4 in range(4):
            out_v[b, pl.ds(o4 * 16, 16)] = zero

        def body(c, carry):
            cwv = [cw_v[b, k, pl.ds(c * 16, 16)] for k in range(_N)]
            for j in range(16):
                i = c * 16 + j
                ck = [cwv[k][j] for k in range(_N)]
                acc = [None] * 4
                for k in range(_N):
                    for m2 in range(2):
                        vi = g[i, pl.ds(16 * (2 * k + m2), 16)]
                        ve = lax.bitcast_convert_type(vi << 16, jnp.float32)
                        vo = lax.bitcast_convert_type(
                            vi & jnp.int32(-65536), jnp.float32)
                        pe = ck[k] * ve
                        po = ck[k] * vo
                        ge, go = 2 * m2, 2 * m2 + 1
                        acc[ge] = pe if acc[ge] is None else acc[ge] + pe
                        acc[go] = po if acc[go] is None else acc[go] + po
                for o4 in range(4):
                    plsc.addupdate(out_v.at[b, pl.ds(o4 * 16, 16)], acc[o4])
            return carry

        lax.fori_loop(0, _CH, body, 0)

    for b0 in range(_NBUF - 1):
        issue(b0, b0)

    @pl.loop(0, _BPW // _NBUF)
    def _main(q):
        b0 = q * _NBUF
        for r in range(_NBUF):
            b = b0 + r
            wait(b, r)
            nxt = b + (_NBUF - 1)

            @pl.when(nxt < _BPW)
            def _():
                issue(nxt, (r + _NBUF - 1) % _NBUF)

            compute(b, r)

    pltpu.sync_copy(out_v, out_hbm.at[pl.ds(base, _BPW)])


def _make_call():
    def body(x_hbm, t_hbm, out_hbm, x_v, idx_v, cw_v,
             g0, g1, g2, g3, out_v, s0, s1, s2, s3):
        _sc_body(x_hbm, t_hbm, out_hbm, x_v, idx_v, cw_v,
                 (g0, g1, g2, g3), out_v, (s0, s1, s2, s3))

    return pl.kernel(
        body,
        out_type=jax.ShapeDtypeStruct((_B, _OUT), jnp.float32),
        mesh=plsc.VectorSubcoreMesh(core_axis_name="c", subcore_axis_name="s"),
        scratch_types=[
            pltpu.VMEM((_BPW, _IN), jnp.float32),
            pltpu.VMEM((_BPW, _IN), jnp.int32),
            pltpu.VMEM((_BPW, _N, _IN), jnp.float32),
            pltpu.VMEM((_IN, _ROW // 2), jnp.int32),
            pltpu.VMEM((_IN, _ROW // 2), jnp.int32),
            pltpu.VMEM((_IN, _ROW // 2), jnp.int32),
            pltpu.VMEM((_IN, _ROW // 2), jnp.int32),
            pltpu.VMEM((_BPW, _OUT), jnp.float32),
            pltpu.SemaphoreType.DMA,
            pltpu.SemaphoreType.DMA,
            pltpu.SemaphoreType.DMA,
            pltpu.SemaphoreType.DMA,
        ],
    )


_sc_call = _make_call()


def kernel(x, w):
    table = _tc_table(w, _sel_matrix())
    return _sc_call(x, table)


# speedup vs baseline: 73.3546x; 1.0374x over previous
import jax
import jax.numpy as jnp
from jax import lax
from jax.experimental import pallas as pl
from jax.experimental.pallas import tpu as pltpu
from jax.experimental.pallas import tpu_sc as plsc

_N = 4
_SEG = 64
_IN = 128
_OUT = 64
_B = 1024
_ROW = _N * _OUT
_NC = 2
_NS = 16
_NW = _NC * _NS
_BPW = _B // _NW
_CH = _IN // 16
_NBUF = 4


_IPB = 8


def _tc_build(w_ref, sel_ref, out_ref):
    for di in range(_IPB):
        w_i = w_ref[:, di, :]
        a = lax.dot_general(sel_ref[...], w_i,
                            (((1,), (1,)), ((), ())),
                            preferred_element_type=jnp.float32)
        b32 = lax.bitcast_convert_type(a, jnp.int32)
        rb = b32 + jnp.int32(0x7FFF) + ((b32 >> 16) & 1)
        bfb = lax.shift_right_logical(rb, 16)
        for m in range(8):
            k, m2 = divmod(m, 2)
            ak = bfb[64 * k : 64 * (k + 1), :]
            lo = ak[:, 32 * m2 : 32 * m2 + 16]
            hi = ak[:, 32 * m2 + 16 : 32 * m2 + 32]
            out_ref[pl.ds(di * _SEG, _SEG), 16 * m : 16 * (m + 1)] = (
                hi << 16) | lo


_tc_table = pl.pallas_call(
    _tc_build,
    grid=(_IN // _IPB,),
    in_specs=[
        pl.BlockSpec((_OUT, _IPB, 193), lambda i: (0, i, 0)),
        pl.BlockSpec((_N * _SEG, 193), lambda i: (0, 0)),
    ],
    out_specs=pl.BlockSpec((_IPB * _SEG, _ROW // 2), lambda i: (i, 0)),
    out_shape=jax.ShapeDtypeStruct((_IN * _SEG, _ROW // 2), jnp.int32),
)


def _sel_matrix():
    l = jnp.arange(193, dtype=jnp.int32)[None, :]
    ks = jnp.arange(_N * _SEG, dtype=jnp.int32)
    tgt = (ks % _SEG) * 3 + ks // _SEG
    return (l == tgt[:, None]).astype(jnp.float32)


def _sc_body(x_hbm, t_hbm, out_hbm, x_v, idx_v, cw_v, gb, out_v, sems):
    wid = lax.axis_index("s") * _NC + lax.axis_index("c")
    base = wid * _BPW
    pltpu.sync_copy(x_hbm.at[pl.ds(base, _BPW)], x_v)

    ivec64 = lax.iota(jnp.int32, 16) * _SEG

    @pl.loop(0, _BPW)
    def _prep(b):
        for c in range(_CH):
            sl = pl.ds(c * 16, 16)
            xv = x_v[b, sl]
            u = 32.0 * xv + 32.0
            seg = jnp.minimum(jnp.maximum(u.astype(jnp.int32), 0), _SEG - 1)
            idx_v[b, sl] = ivec64 + (c * 16 * _SEG) + seg
            t = 2.0 * (u - seg.astype(jnp.float32)) - 1.0
            ta = t * t - (1.0 / 9.0)
            tb = t * t - 1.0
            cw_v[b, 0, sl] = (-9.0 / 16.0) * ta * (t - 1.0)
            cw_v[b, 1, sl] = (27.0 / 16.0) * tb * (t - 1.0 / 3.0)
            cw_v[b, 2, sl] = (-27.0 / 16.0) * tb * (t + 1.0 / 3.0)
            cw_v[b, 3, sl] = (9.0 / 16.0) * ta * (t + 1.0)

    def issue(b, which):
        pltpu.async_copy(t_hbm.at[idx_v.at[b]], gb[which], sems[which])

    def wait(b, which):
        pltpu.make_async_copy(t_hbm.at[idx_v.at[b]], gb[which], sems[which]).wait()

    def compute(b, which):
        g = gb[which]
        zero = jnp.zeros((16,), jnp.float32)
        for o4 in range(4):
            out_v[b, pl.ds(o4 * 16, 16)] = zero

        def body(c, carry):
            cwv = [cw_v[b, k, pl.ds(c * 16, 16)] for k in range(_N)]
            for j in range(16):
                i = c * 16 + j
                ck = [cwv[k][j] for k in range(_N)]
                acc = [None] * 4
                for k in range(_N):
                    for m2 in range(2):
                        vi = g[i, pl.ds(16 * (2 * k + m2), 16)]
                        ve = lax.bitcast_convert_type(vi << 16, jnp.float32)
                        vo = lax.bitcast_convert_type(vi, jnp.float32)
                        pe = ck[k] * ve
                        po = ck[k] * vo
                        ge, go = 2 * m2, 2 * m2 + 1
                        acc[ge] = pe if acc[ge] is None else acc[ge] + pe
                        acc[go] = po if acc[go] is None else acc[go] + po
                for o4 in range(4):
                    plsc.addupdate(out_v.at[b, pl.ds(o4 * 16, 16)], acc[o4])
            return carry

        lax.fori_loop(0, _CH, body, 0)

    for b0 in range(_NBUF - 1):
        issue(b0, b0)

    @pl.loop(0, _BPW // _NBUF)
    def _main(q):
        b0 = q * _NBUF
        for r in range(_NBUF):
            b = b0 + r
            wait(b, r)
            nxt = b + (_NBUF - 1)

            @pl.when(nxt < _BPW)
            def _():
                issue(nxt, (r + _NBUF - 1) % _NBUF)

            compute(b, r)

    pltpu.sync_copy(out_v, out_hbm.at[pl.ds(base, _BPW)])


def _make_call():
    def body(x_hbm, t_hbm, out_hbm, x_v, idx_v, cw_v,
             g0, g1, g2, g3, out_v, s0, s1, s2, s3):
        _sc_body(x_hbm, t_hbm, out_hbm, x_v, idx_v, cw_v,
                 (g0, g1, g2, g3), out_v, (s0, s1, s2, s3))

    return pl.kernel(
        body,
        out_type=jax.ShapeDtypeStruct((_B, _OUT), jnp.float32),
        mesh=plsc.VectorSubcoreMesh(core_axis_name="c", subcore_axis_name="s"),
        scratch_types=[
            pltpu.VMEM((_BPW, _IN), jnp.float32),
            pltpu.VMEM((_BPW, _IN), jnp.int32),
            pltpu.VMEM((_BPW, _N, _IN), jnp.float32),
            pltpu.VMEM((_IN, _ROW // 2), jnp.int32),
            pltpu.VMEM((_IN, _ROW // 2), jnp.int32),
            pltpu.VMEM((_IN, _ROW // 2), jnp.int32),
            pltpu.VMEM((_IN, _ROW // 2), jnp.int32),
            pltpu.VMEM((_BPW, _OUT), jnp.float32),
            pltpu.SemaphoreType.DMA,
            pltpu.SemaphoreType.DMA,
            pltpu.SemaphoreType.DMA,
            pltpu.SemaphoreType.DMA,
        ],
    )


_sc_call = _make_call()


def kernel(x, w):
    table = _tc_table(w, _sel_matrix())
    return _sc_call(x, table)


# speedup vs baseline: 73.7608x; 1.0055x over previous
import jax
import jax.numpy as jnp
from jax import lax
from jax.experimental import pallas as pl
from jax.experimental.pallas import tpu as pltpu
from jax.experimental.pallas import tpu_sc as plsc

_N = 4
_SEG = 64
_IN = 128
_OUT = 64
_B = 1024
_ROW = _N * _OUT
_NC = 2
_NS = 16
_NW = _NC * _NS
_BPW = _B // _NW
_CH = _IN // 16
_NBUF = 4


_IPB = 8


def _tc_build(w_ref, sel_ref, out_ref):
    wb = w_ref[...].astype(jnp.bfloat16)
    sel = sel_ref[...]
    avals = [lax.dot_general(sel, wb[:, di, :],
                             (((1,), (1,)), ((), ())),
                             preferred_element_type=jnp.float32)
             for di in range(_IPB)]
    for di in range(_IPB):
        b32 = lax.bitcast_convert_type(avals[di], jnp.int32)
        bfb = lax.shift_right_logical(b32, 16)
        for k in range(_N):
            ak = bfb[64 * k : 64 * (k + 1), :]
            lo = jnp.concatenate([ak[:, 0:16], ak[:, 32:48]], axis=1)
            hi = jnp.concatenate([ak[:, 16:32], ak[:, 48:64]], axis=1)
            out_ref[pl.ds(di * _SEG, _SEG), 32 * k : 32 * (k + 1)] = (
                hi << 16) | lo


_tc_table = pl.pallas_call(
    _tc_build,
    grid=(_IN // _IPB,),
    in_specs=[
        pl.BlockSpec((_OUT, _IPB, 193), lambda i: (0, i, 0)),
        pl.BlockSpec((_N * _SEG, 193), lambda i: (0, 0)),
    ],
    out_specs=pl.BlockSpec((_IPB * _SEG, _ROW // 2), lambda i: (i, 0)),
    out_shape=jax.ShapeDtypeStruct((_IN * _SEG, _ROW // 2), jnp.int32),
)


def _sel_matrix():
    l = jnp.arange(193, dtype=jnp.int32)[None, :]
    ks = jnp.arange(_N * _SEG, dtype=jnp.int32)
    tgt = (ks % _SEG) * 3 + ks // _SEG
    return (l == tgt[:, None]).astype(jnp.bfloat16)


def _sc_body(x_hbm, t_hbm, out_hbm, x_v, idx_v, cw_v, gb, out_v, sems):
    wid = lax.axis_index("s") * _NC + lax.axis_index("c")
    base = wid * _BPW
    pltpu.sync_copy(x_hbm.at[pl.ds(base, _BPW)], x_v)

    ivec64 = lax.iota(jnp.int32, 16) * _SEG

    @pl.loop(0, _BPW)
    def _prep(b):
        for c in range(_CH):
            sl = pl.ds(c * 16, 16)
            xv = x_v[b, sl]
            u = 32.0 * xv + 32.0
            seg = jnp.minimum(jnp.maximum(u.astype(jnp.int32), 0), _SEG - 1)
            idx_v[b, sl] = ivec64 + (c * 16 * _SEG) + seg
            t = 2.0 * (u - seg.astype(jnp.float32)) - 1.0
            ta = t * t - (1.0 / 9.0)
            tb = t * t - 1.0
            cw_v[b, 0, sl] = (-9.0 / 16.0) * ta * (t - 1.0)
            cw_v[b, 1, sl] = (27.0 / 16.0) * tb * (t - 1.0 / 3.0)
            cw_v[b, 2, sl] = (-27.0 / 16.0) * tb * (t + 1.0 / 3.0)
            cw_v[b, 3, sl] = (9.0 / 16.0) * ta * (t + 1.0)

    def issue(b, which):
        pltpu.async_copy(t_hbm.at[idx_v.at[b]], gb[which], sems[which])

    def wait(b, which):
        pltpu.make_async_copy(t_hbm.at[idx_v.at[b]], gb[which], sems[which]).wait()

    def compute(b, which):
        g = gb[which]
        zero = jnp.zeros((16,), jnp.float32)
        for o4 in range(4):
            out_v[b, pl.ds(o4 * 16, 16)] = zero

        def body(c, carry):
            cwv = [cw_v[b, k, pl.ds(c * 16, 16)] for k in range(_N)]
            for j in range(16):
                i = c * 16 + j
                ck = [cwv[k][j] for k in range(_N)]
                acc = [None] * 4
                for k in range(_N):
                    for m2 in range(2):
                        vi = g[i, pl.ds(16 * (2 * k + m2), 16)]
                        ve = lax.bitcast_convert_type(vi << 16, jnp.float32)
                        vo = lax.bitcast_convert_type(vi, jnp.float32)
                        pe = ck[k] * ve
                        po = ck[k] * vo
                        ge, go = 2 * m2, 2 * m2 + 1
                        acc[ge] = pe if acc[ge] is None else acc[ge] + pe
                        acc[go] = po if acc[go] is None else acc[go] + po
                for o4 in range(4):
                    plsc.addupdate(out_v.at[b, pl.ds(o4 * 16, 16)], acc[o4])
            return carry

        lax.fori_loop(0, _CH, body, 0)

    for b0 in range(_NBUF - 1):
        issue(b0, b0)

    @pl.loop(0, _BPW // _NBUF)
    def _main(q):
        b0 = q * _NBUF
        for r in range(_NBUF):
            b = b0 + r
            wait(b, r)
            nxt = b + (_NBUF - 1)

            @pl.when(nxt < _BPW)
            def _():
                issue(nxt, (r + _NBUF - 1) % _NBUF)

            compute(b, r)

    pltpu.sync_copy(out_v, out_hbm.at[pl.ds(base, _BPW)])


def _make_call():
    def body(x_hbm, t_hbm, out_hbm, x_v, idx_v, cw_v,
             g0, g1, g2, g3, out_v, s0, s1, s2, s3):
        _sc_body(x_hbm, t_hbm, out_hbm, x_v, idx_v, cw_v,
                 (g0, g1, g2, g3), out_v, (s0, s1, s2, s3))

    return pl.kernel(
        body,
        out_type=jax.ShapeDtypeStruct((_B, _OUT), jnp.float32),
        mesh=plsc.VectorSubcoreMesh(core_axis_name="c", subcore_axis_name="s"),
        scratch_types=[
            pltpu.VMEM((_BPW, _IN), jnp.float32),
            pltpu.VMEM((_BPW, _IN), jnp.int32),
            pltpu.VMEM((_BPW, _N, _IN), jnp.float32),
            pltpu.VMEM((_IN, _ROW // 2), jnp.int32),
            pltpu.VMEM((_IN, _ROW // 2), jnp.int32),
            pltpu.VMEM((_IN, _ROW // 2), jnp.int32),
            pltpu.VMEM((_IN, _ROW // 2), jnp.int32),
            pltpu.VMEM((_BPW, _OUT), jnp.float32),
            pltpu.SemaphoreType.DMA,
            pltpu.SemaphoreType.DMA,
            pltpu.SemaphoreType.DMA,
            pltpu.SemaphoreType.DMA,
        ],
    )


_sc_call = _make_call()


def kernel(x, w):
    table = _tc_table(w, _sel_matrix())
    return _sc_call(x, table)


# speedup vs baseline: 76.1029x; 1.0318x over previous
import jax
import jax.numpy as jnp
from jax import lax
from jax.experimental import pallas as pl
from jax.experimental.pallas import tpu as pltpu
from jax.experimental.pallas import tpu_sc as plsc

_N = 4
_SEG = 64
_IN = 128
_OUT = 64
_B = 1024
_ROW = _N * _OUT
_NC = 2
_NS = 16
_NW = _NC * _NS
_BPW = _B // _NW
_CH = _IN // 16
_NBUF = 4


_IPB = 8


def _tc_build(w_ref, sel_ref, out_ref):
    wb = w_ref[...].astype(jnp.bfloat16)
    sel = sel_ref[...]
    avals = [lax.dot_general(sel, wb[:, di, :],
                             (((1,), (1,)), ((), ())),
                             preferred_element_type=jnp.float32)
             for di in range(_IPB)]
    for di in range(_IPB):
        b32 = lax.bitcast_convert_type(avals[di], jnp.int32)
        bfb = lax.shift_right_logical(b32, 16)
        for k in range(_N):
            ak = bfb[64 * k : 64 * (k + 1), :]
            lo = jnp.concatenate([ak[:, 0:16], ak[:, 32:48]], axis=1)
            hi = jnp.concatenate([ak[:, 16:32], ak[:, 48:64]], axis=1)
            out_ref[pl.ds(di * _SEG, _SEG), 32 * k : 32 * (k + 1)] = (
                hi << 16) | lo


_tc_table = pl.pallas_call(
    _tc_build,
    grid=(_IN // _IPB,),
    in_specs=[
        pl.BlockSpec((_OUT, _IPB, 193), lambda i: (0, i, 0)),
        pl.BlockSpec((_N * _SEG, 193), lambda i: (0, 0)),
    ],
    out_specs=pl.BlockSpec((_IPB * _SEG, _ROW // 2), lambda i: (i, 0)),
    out_shape=jax.ShapeDtypeStruct((_IN * _SEG, _ROW // 2), jnp.int32),
)


def _sel_matrix():
    l = jnp.arange(193, dtype=jnp.int32)[None, :]
    ks = jnp.arange(_N * _SEG, dtype=jnp.int32)
    tgt = (ks % _SEG) * 3 + ks // _SEG
    return (l == tgt[:, None]).astype(jnp.bfloat16)


def _sc_body(x_hbm, t_hbm, out_hbm, x_v, idx_v, cw_v, gb, out_v, sems):
    wid = lax.axis_index("s") * _NC + lax.axis_index("c")
    base = wid * _BPW
    pltpu.sync_copy(x_hbm.at[pl.ds(base, _BPW)], x_v)

    ivec64 = lax.iota(jnp.int32, 16) * _SEG

    def _prep_row(b):
        for c in range(_CH):
            sl = pl.ds(c * 16, 16)
            xv = x_v[b, sl]
            u = 32.0 * xv + 32.0
            seg = jnp.minimum(jnp.maximum(u.astype(jnp.int32), 0), _SEG - 1)
            idx_v[b, sl] = ivec64 + (c * 16 * _SEG) + seg
            t = 2.0 * (u - seg.astype(jnp.float32)) - 1.0
            ta = t * t - (1.0 / 9.0)
            tb = t * t - 1.0
            cw_v[b, 0, sl] = (-9.0 / 16.0) * ta * (t - 1.0)
            cw_v[b, 1, sl] = (27.0 / 16.0) * tb * (t - 1.0 / 3.0)
            cw_v[b, 2, sl] = (-27.0 / 16.0) * tb * (t + 1.0 / 3.0)
            cw_v[b, 3, sl] = (9.0 / 16.0) * ta * (t + 1.0)

    def issue(b, which):
        pltpu.async_copy(t_hbm.at[idx_v.at[b]], gb[which], sems[which])

    def wait(b, which):
        pltpu.make_async_copy(t_hbm.at[idx_v.at[b]], gb[which], sems[which]).wait()

    def compute(b, which):
        g = gb[which]
        zero = jnp.zeros((16,), jnp.float32)
        for o4 in range(4):
            out_v[b, pl.ds(o4 * 16, 16)] = zero

        def body(c, carry):
            cwv = [cw_v[b, k, pl.ds(c * 16, 16)] for k in range(_N)]
            for j in range(16):
                i = c * 16 + j
                ck = [cwv[k][j] for k in range(_N)]
                acc = [None] * 4
                for k in range(_N):
                    for m2 in range(2):
                        vi = g[i, pl.ds(16 * (2 * k + m2), 16)]
                        ve = lax.bitcast_convert_type(vi << 16, jnp.float32)
                        vo = lax.bitcast_convert_type(vi, jnp.float32)
                        pe = ck[k] * ve
                        po = ck[k] * vo
                        ge, go = 2 * m2, 2 * m2 + 1
                        acc[ge] = pe if acc[ge] is None else acc[ge] + pe
                        acc[go] = po if acc[go] is None else acc[go] + po
                for o4 in range(4):
                    plsc.addupdate(out_v.at[b, pl.ds(o4 * 16, 16)], acc[o4])
            return carry

        lax.fori_loop(0, _CH, body, 0)

    for b0 in range(_NBUF - 1):
        _prep_row(b0)
    for b0 in range(_NBUF - 1):
        issue(b0, b0)

    @pl.loop(_NBUF - 1, _BPW)
    def _prep(b):
        _prep_row(b)

    @pl.loop(0, _BPW // _NBUF)
    def _main(q):
        b0 = q * _NBUF
        for r in range(_NBUF):
            b = b0 + r
            wait(b, r)
            nxt = b + (_NBUF - 1)

            @pl.when(nxt < _BPW)
            def _():
                issue(nxt, (r + _NBUF - 1) % _NBUF)

            compute(b, r)

    pltpu.sync_copy(out_v, out_hbm.at[pl.ds(base, _BPW)])


def _make_call():
    def body(x_hbm, t_hbm, out_hbm, x_v, idx_v, cw_v,
             g0, g1, g2, g3, out_v, s0, s1, s2, s3):
        _sc_body(x_hbm, t_hbm, out_hbm, x_v, idx_v, cw_v,
                 (g0, g1, g2, g3), out_v, (s0, s1, s2, s3))

    return pl.kernel(
        body,
        out_type=jax.ShapeDtypeStruct((_B, _OUT), jnp.float32),
        mesh=plsc.VectorSubcoreMesh(core_axis_name="c", subcore_axis_name="s"),
        scratch_types=[
            pltpu.VMEM((_BPW, _IN), jnp.float32),
            pltpu.VMEM((_BPW, _IN), jnp.int32),
            pltpu.VMEM((_BPW, _N, _IN), jnp.float32),
            pltpu.VMEM((_IN, _ROW // 2), jnp.int32),
            pltpu.VMEM((_IN, _ROW // 2), jnp.int32),
            pltpu.VMEM((_IN, _ROW // 2), jnp.int32),
            pltpu.VMEM((_IN, _ROW // 2), jnp.int32),
            pltpu.VMEM((_BPW, _OUT), jnp.float32),
            pltpu.SemaphoreType.DMA,
            pltpu.SemaphoreType.DMA,
            pltpu.SemaphoreType.DMA,
            pltpu.SemaphoreType.DMA,
        ],
    )


_sc_call = _make_call()


def kernel(x, w):
    table = _tc_table(w, _sel_matrix())
    return _sc_call(x, table)
